# Initial kernel scaffold; baseline (speedup 1.0000x reference)
import jax
import jax.numpy as jnp
from jax.experimental import pallas as pl


def kernel(x, table):
    raise NotImplementedError("write your pallas kernel here")


# speedup vs baseline: 2.4194x; 2.4194x over previous
import functools
import math

import jax
import jax.numpy as jnp
from jax import lax
from jax.experimental import pallas as pl
from jax.experimental.pallas import tpu as pltpu
from jax.experimental.pallas import tpu_sc as plsc

_B = 4096
_L = 50
_DIM = 128
_SCALE = math.sqrt(_DIM)

_N = _B * _L

_info = plsc.get_sparse_core_info()
_NC = _info.num_cores
_NS = _info.num_subcores
_NW = _NC * _NS
_PER_W = _N // _NW
_G = 128
_NG = _PER_W // _G

_mesh = plsc.VectorSubcoreMesh(core_axis_name="c", subcore_axis_name="s")


@functools.partial(
    pl.kernel,
    mesh=_mesh,
    out_type=jax.ShapeDtypeStruct((_N, _DIM), jnp.float32),
    scratch_types=[
        pltpu.VMEM((_NG, _G), jnp.int32),
        pltpu.VMEM((_G, _DIM), jnp.float32),
        pltpu.SemaphoreType.DMA,
    ],
)
def _gather_scale(table_hbm, idx_hbm, out_hbm, idx_v, rows_v, sem):
    wid = lax.axis_index("s") * _NC + lax.axis_index("c")
    base = wid * _PER_W
    pltpu.sync_copy(idx_hbm.at[wid], idx_v)

    def group_body(g, carry):
        pltpu.async_copy(table_hbm.at[idx_v.at[g]], rows_v, sem).wait()

        def row_body(r, c):
            for j in range(_DIM // 16):
                sl = pl.ds(j * 16, 16)
                rows_v[r, sl] = rows_v[r, sl] * _SCALE
            return c

        lax.fori_loop(0, _G, row_body, 0)

        pltpu.sync_copy(rows_v, out_hbm.at[pl.ds(base + g * _G, _G)])
        return carry

    lax.fori_loop(0, _NG, group_body, 0)


def kernel(x, table):
    idx = x.reshape(_NW, _NG, _G)
    out = _gather_scale(table, idx)
    return out.reshape(_B, _L, _DIM)


# speedup vs baseline: 2.9600x; 1.2234x over previous
import functools
import math

import jax
import jax.numpy as jnp
from jax import lax
from jax.experimental import pallas as pl
from jax.experimental.pallas import tpu as pltpu
from jax.experimental.pallas import tpu_sc as plsc

_B = 4096
_L = 50
_DIM = 128
_SCALE = math.sqrt(_DIM)

_N = _B * _L

_info = plsc.get_sparse_core_info()
_NC = _info.num_cores
_NS = _info.num_subcores
_NW = _NC * _NS
_PER_W = _N // _NW
_G = 128
_NG = _PER_W // _G
_NBUF = 5
_LA = 3

_mesh = plsc.VectorSubcoreMesh(core_axis_name="c", subcore_axis_name="s")


@functools.partial(
    pl.kernel,
    mesh=_mesh,
    out_type=jax.ShapeDtypeStruct((_N, _DIM), jnp.float32),
    scratch_types=[
        pltpu.VMEM((_NG, _G), jnp.int32),
        pltpu.VMEM((_NBUF, _G, _DIM), jnp.float32),
    ]
    + [pltpu.SemaphoreType.DMA] * (2 * _NBUF),
)
def _gather_scale(table_hbm, idx_hbm, out_hbm, idx_v, rows_v, *sems):
    sem_g = sems[:_NBUF]
    sem_s = sems[_NBUF:]
    wid = lax.axis_index("s") * _NC + lax.axis_index("c")
    base = wid * _PER_W
    pltpu.sync_copy(idx_hbm.at[wid], idx_v)

    def start_gather(g, b):
        pltpu.make_async_copy(
            table_hbm.at[idx_v.at[g]], rows_v.at[b], sem_g[b]
        ).start()

    def scatter_copy(g, b):
        return pltpu.make_async_copy(
            rows_v.at[b], out_hbm.at[pl.ds(base + g * _G, _G)], sem_s[b]
        )

    for g in range(_LA):
        start_gather(g, g)

    def round_body(r, carry):
        for b in range(_NBUF):
            g = r * _NBUF + b
            pltpu.make_async_copy(
                table_hbm.at[idx_v.at[g]], rows_v.at[b], sem_g[b]
            ).wait()

            def row_body(row, c, _b=b):
                for j in range(_DIM // 16):
                    sl = pl.ds(j * 16, 16)
                    rows_v[_b, row, sl] = rows_v[_b, row, sl] * _SCALE
                return c

            lax.fori_loop(0, _G, row_body, 0)

            scatter_copy(g, b).start()

            bf = (b + _LA) % _NBUF
            gf = g + _LA

            @pl.when(gf < _NG)
            def _():
                @pl.when(gf >= _NBUF)
                def _():
                    scatter_copy(gf - _NBUF, bf).wait()

                start_gather(gf, bf)

        return carry

    lax.fori_loop(0, _NG // _NBUF, round_body, 0)

    for b in range(_NBUF):
        scatter_copy(_NG - _NBUF + b, b).wait()


def kernel(x, table):
    idx = x.reshape(_NW, _NG, _G)
    out = _gather_scale(table, idx)
    return out.reshape(_B, _L, _DIM)


# speedup vs baseline: 5.2807x; 1.7840x over previous
import functools
import math

import jax
import jax.numpy as jnp
from jax import lax
from jax.experimental import pallas as pl
from jax.experimental.pallas import tpu as pltpu
from jax.experimental.pallas import tpu_sc as plsc

_B = 4096
_L = 50
_DIM = 128
_SCALE = math.sqrt(_DIM)

_info = plsc.get_sparse_core_info()
_NC = _info.num_cores
_NS = _info.num_subcores
_NW = _NC * _NS
_BPW = _B // _NW
_NBUF = 8
_LA = 4

_mesh = plsc.VectorSubcoreMesh(core_axis_name="c", subcore_axis_name="s")


@functools.partial(
    pl.kernel,
    mesh=_mesh,
    out_type=jax.ShapeDtypeStruct((_B, _L, _DIM), jnp.float32),
    scratch_types=[
        pltpu.VMEM((_BPW, _L), jnp.int32),
        pltpu.VMEM((_NBUF, _L, _DIM), jnp.float32),
    ]
    + [pltpu.SemaphoreType.DMA] * (2 * _NBUF),
)
def _gather_scale(table_hbm, idx_hbm, out_hbm, idx_v, rows_v, *sems):
    sem_g = sems[:_NBUF]
    sem_s = sems[_NBUF:]
    wid = lax.axis_index("s") * _NC + lax.axis_index("c")
    base = wid * _BPW
    pltpu.sync_copy(idx_hbm.at[pl.ds(base, _BPW)], idx_v)

    def gather_copy(g, b):
        return pltpu.make_async_copy(
            table_hbm.at[idx_v.at[g]], rows_v.at[b], sem_g[b]
        )

    def scatter_copy(g, b):
        return pltpu.make_async_copy(
            rows_v.at[b], out_hbm.at[base + g], sem_s[b]
        )

    for g in range(_LA):
        gather_copy(g, g).start()

    def round_body(r, carry):
        for b in range(_NBUF):
            g = r * _NBUF + b
            gather_copy(g, b).wait()

            def row_body(row, c, _b=b):
                for j in range(_DIM // 16):
                    sl = pl.ds(j * 16, 16)
                    rows_v[_b, row, sl] = rows_v[_b, row, sl] * _SCALE
                return c

            lax.fori_loop(0, _L, row_body, 0)

            scatter_copy(g, b).start()

            bf = (b + _LA) % _NBUF
            gf = g + _LA

            @pl.when(gf < _BPW)
            def _():
                @pl.when(gf >= _NBUF)
                def _():
                    scatter_copy(gf - _NBUF, bf).wait()

                gather_copy(gf, bf).start()

        return carry

    lax.fori_loop(0, _BPW // _NBUF, round_body, 0)

    for b in range(_NBUF):
        scatter_copy(_BPW - _NBUF + b, b).wait()


def kernel(x, table):
    return _gather_scale(table, x)
